# Initial kernel scaffold; baseline (speedup 1.0000x reference)
import functools

import jax
import jax.numpy as jnp
from jax import lax
from jax.experimental import pallas as pl
from jax.experimental.pallas import tpu as pltpu
from jax.experimental.pallas import tpu_sc as plsc

NC = 2
NS = 16
NW = NC * NS
IW = 128


@functools.cache
def _build(V, D, N):
    ROWS = N // IW
    RPW = ROWS // NW
    G = 4
    NBLK = RPW // G
    BLK = G * IW

    mesh = plsc.VectorSubcoreMesh(core_axis_name="c", subcore_axis_name="s")

    @functools.partial(
        pl.kernel,
        mesh=mesh,
        out_type=jax.ShapeDtypeStruct((N, D), jnp.float32),
        scratch_types=[
            pltpu.VMEM((RPW, IW), jnp.int32),
            pltpu.VMEM((BLK, D), jnp.float32),
            pltpu.VMEM((BLK, D), jnp.float32),
            pltpu.SemaphoreType.DMA,
            pltpu.SemaphoreType.DMA,
        ],
    )
    def emb(table_hbm, idx_hbm, out_hbm, idx_v, rows0, rows1, sem0, sem1):
        wid = lax.axis_index("s") * NC + lax.axis_index("c")
        row0 = wid * RPW
        pltpu.sync_copy(idx_hbm.at[pl.ds(row0, RPW)], idx_v)

        def body(i, carry):
            b0 = 2 * i
            b1 = b0 + 1
            cps = []
            for buf, sem, b in ((rows0, sem0, b0), (rows1, sem1, b1)):
                for g in range(G):
                    cps.append(pltpu.async_copy(
                        table_hbm.at[idx_v.at[b * G + g]],
                        buf.at[pl.ds(g * IW, IW)],
                        sem,
                    ))
            for buf, b, lo in ((rows0, b0, 0), (rows1, b1, G)):
                for cp in cps[lo:lo + G]:
                    cp.wait()
                base = (row0 + b * G) * IW
                pltpu.sync_copy(buf, out_hbm.at[pl.ds(base, BLK)])
            return carry

        lax.fori_loop(0, NBLK // 2, body, 0)

    return emb


def kernel(word, table):
    B, L = word.shape
    V, D = table.shape
    N = B * L
    idx = word.reshape(N // IW, IW)
    out = _build(V, D, N)(table, idx)
    return out.reshape(B, L, D)


# speedup vs baseline: 3.5797x; 3.5797x over previous
import functools

import jax
import jax.numpy as jnp
from jax import lax
from jax.experimental import pallas as pl
from jax.experimental.pallas import tpu as pltpu
from jax.experimental.pallas import tpu_sc as plsc

NC = 2
NS = 16
NW = NC * NS
IW = 128


DP = 56


@functools.cache
def _build(V, D, N):
    ROWS = N // IW
    RPW = ROWS // NW
    G = 4
    NBLK = RPW // G
    BLK = G * IW

    mesh = plsc.VectorSubcoreMesh(core_axis_name="c", subcore_axis_name="s")

    @functools.partial(
        pl.kernel,
        mesh=mesh,
        compiler_params=pltpu.CompilerParams(use_tc_tiling_on_sc=False),
        out_type=jax.ShapeDtypeStruct((N, DP), jnp.float32),
        scratch_types=[
            pltpu.VMEM((RPW, IW), jnp.int32),
            pltpu.VMEM((BLK, DP), jnp.float32),
            pltpu.VMEM((BLK, DP), jnp.float32),
            pltpu.SemaphoreType.DMA,
            pltpu.SemaphoreType.DMA,
        ],
    )
    def emb(table_hbm, idx_hbm, out_hbm, idx_v, rows0, rows1, sem0, sem1):
        wid = lax.axis_index("s") * NC + lax.axis_index("c")
        row0 = wid * RPW
        pltpu.sync_copy(idx_hbm.at[pl.ds(row0, RPW)], idx_v)

        def body(i, carry):
            b0 = 2 * i
            b1 = b0 + 1
            cps = []
            for buf, sem, b in ((rows0, sem0, b0), (rows1, sem1, b1)):
                for g in range(G):
                    cps.append(pltpu.async_copy(
                        table_hbm.at[idx_v.at[b * G + g]],
                        buf.at[pl.ds(g * IW, IW)],
                        sem,
                    ))
            for buf, b, lo in ((rows0, b0, 0), (rows1, b1, G)):
                for cp in cps[lo:lo + G]:
                    cp.wait()
                base = (row0 + b * G) * IW
                pltpu.sync_copy(buf, out_hbm.at[pl.ds(base, BLK)])
            return carry

        lax.fori_loop(0, NBLK // 2, body, 0)

    return emb


def kernel(word, table):
    B, L = word.shape
    V, D = table.shape
    N = B * L
    idx = word.reshape(N // IW, IW)
    table_p = jnp.pad(table, ((0, 0), (0, DP - D)))
    out = _build(V, D, N)(table_p, idx)
    return out[:, :D].reshape(B, L, D)


# speedup vs baseline: 6.6917x; 1.8694x over previous
import functools

import jax
import jax.numpy as jnp
from jax import lax
from jax.experimental import pallas as pl
from jax.experimental.pallas import tpu as pltpu
from jax.experimental.pallas import tpu_sc as plsc

NC = 2
NS = 16
NW = NC * NS
IW = 128


DP = 56


@functools.cache
def _build(V, D, N):
    ROWS = N // IW
    RPW = ROWS // NW
    G = 4
    NBLK = RPW // G
    BLK = G * IW

    mesh = plsc.VectorSubcoreMesh(core_axis_name="c", subcore_axis_name="s")

    @functools.partial(
        pl.kernel,
        mesh=mesh,
        compiler_params=pltpu.CompilerParams(use_tc_tiling_on_sc=False),
        out_type=jax.ShapeDtypeStruct((N, 128), jnp.float32),
        scratch_types=[
            pltpu.VMEM((RPW, IW), jnp.int32),
            pltpu.VMEM((BLK, DP), jnp.float32),
            pltpu.VMEM((BLK, DP), jnp.float32),
            pltpu.SemaphoreType.DMA,
            pltpu.SemaphoreType.DMA,
        ],
    )
    def emb(table_hbm, idx_hbm, out_hbm, idx_v, rows0, rows1, sem0, sem1):
        wid = lax.axis_index("s") * NC + lax.axis_index("c")
        row0 = wid * RPW
        pltpu.sync_copy(idx_hbm.at[pl.ds(row0, RPW)], idx_v)

        def body(i, carry):
            b0 = 2 * i
            b1 = b0 + 1
            cps = []
            for buf, sem, b in ((rows0, sem0, b0), (rows1, sem1, b1)):
                for g in range(G):
                    cps.append(pltpu.async_copy(
                        table_hbm.at[idx_v.at[b * G + g]],
                        buf.at[pl.ds(g * IW, IW)],
                        sem,
                    ))
            for buf, b, lo in ((rows0, b0, 0), (rows1, b1, G)):
                for cp in cps[lo:lo + G]:
                    cp.wait()
                base = (row0 + b * G) * IW
                pltpu.sync_copy(
                    buf, out_hbm.at[pl.ds(base, BLK), pl.ds(0, DP)])
            return carry

        lax.fori_loop(0, NBLK // 2, body, 0)

    return emb


def kernel(word, table):
    B, L = word.shape
    V, D = table.shape
    N = B * L
    idx = word.reshape(N // IW, IW)
    table_p = jnp.pad(table, ((0, 0), (0, DP - D)))
    out = _build(V, D, N)(table_p, idx)
    return out[:, :D].reshape(B, L, D)
